# Initial kernel scaffold; baseline (speedup 1.0000x reference)
import jax
import jax.numpy as jnp
from jax.experimental import pallas as pl


def kernel(inputs, embedding):
    raise NotImplementedError("write your pallas kernel here")


# speedup vs baseline: 1.7218x; 1.7218x over previous
import jax
import jax.numpy as jnp
from jax.experimental import pallas as pl

_EMBED_DIM = 64
_N_EMBED = 1024
_BLOCK_M = 512


def _vq_block(x_ref, emb_ref, embt_ref, idx_ref, quant_ref, codes_ref):
    x = x_ref[...]
    emb = emb_ref[...]
    embt = embt_ref[...]
    x2 = jnp.sum(x * x, axis=1, keepdims=True)
    e2 = jnp.sum(emb * emb, axis=0, keepdims=True)
    dot = jnp.dot(x, emb, preferred_element_type=jnp.float32)
    d = (x2 - 2.0 * dot) + e2
    m = jnp.min(d, axis=1, keepdims=True)
    iota = jax.lax.broadcasted_iota(jnp.int32, d.shape, 1)
    idx = jnp.min(jnp.where(d == m, iota, _N_EMBED), axis=1)
    idx_ref[0, 0, :] = idx
    onehot = (iota == idx[:, None]).astype(jnp.float32)
    q = jnp.dot(onehot, embt, preferred_element_type=jnp.float32)
    qst = x + (q - x)
    quant_ref[...] = qst
    codes_ref[...] = jnp.concatenate([x, q], axis=1)


def kernel(inputs, embedding):
    lead_shape = inputs.shape[:-1]
    flat = inputs.reshape(-1, _EMBED_DIM)
    n_rows = flat.shape[0]
    grid = n_rows // _BLOCK_M
    embt = embedding.T

    idx3, quant, codes = pl.pallas_call(
        _vq_block,
        grid=(grid,),
        in_specs=[
            pl.BlockSpec((_BLOCK_M, _EMBED_DIM), lambda i: (i, 0)),
            pl.BlockSpec((_EMBED_DIM, _N_EMBED), lambda i: (0, 0)),
            pl.BlockSpec((_N_EMBED, _EMBED_DIM), lambda i: (0, 0)),
        ],
        out_specs=[
            pl.BlockSpec((1, 1, _BLOCK_M), lambda i: (i, 0, 0)),
            pl.BlockSpec((_BLOCK_M, _EMBED_DIM), lambda i: (i, 0)),
            pl.BlockSpec((_BLOCK_M, 2 * _EMBED_DIM), lambda i: (i, 0)),
        ],
        out_shape=[
            jax.ShapeDtypeStruct((grid, 1, _BLOCK_M), jnp.int32),
            jax.ShapeDtypeStruct((n_rows, _EMBED_DIM), jnp.float32),
            jax.ShapeDtypeStruct((n_rows, 2 * _EMBED_DIM), jnp.float32),
        ],
    )(flat, embedding, embt)

    quantized = quant.reshape(inputs.shape)
    codes_out = codes.reshape(lead_shape + (2 * _EMBED_DIM,))
    encoding_indices = idx3.reshape(lead_shape)
    return (quantized, codes_out, encoding_indices)
